# Initial kernel scaffold; baseline (speedup 1.0000x reference)
import jax
import jax.numpy as jnp
from jax.experimental import pallas as pl


def kernel(mem, grid_x, grid_y, updates):
    raise NotImplementedError("write your pallas kernel here")


# speedup vs baseline: 48.1152x; 48.1152x over previous
import functools

import jax
import jax.numpy as jnp
from jax import lax
from jax.experimental import pallas as pl
from jax.experimental.pallas import tpu as pltpu
from jax.experimental.pallas import tpu_sc as plsc

NC, NS, L = 2, 16, 16
BASE = 1 << 26
MASK = BASE - 1
ROUNDS = 6


@functools.lru_cache(maxsize=None)
def _build(N, M, H, B):
    NM = N * M
    RCH = B // NS
    GCH = B // (NC * NS)
    GROWS = GCH // 128
    STRIPE = NM // NS
    mesh = plsc.VectorSubcoreMesh(core_axis_name="c", subcore_axis_name="s")

    def body(gx_hbm, gy_hbm, upd_hbm, out_hbm,
             table, gxf, gyf, keys, gbuf, vbuf, act, widx, zbuf, rowbuf,
             sem0, sem1):
        cid = lax.axis_index("c")
        sid = lax.axis_index("s")
        rbase = sid * RCH
        iota = lax.broadcasted_iota(jnp.int32, (L,), 0)

        pltpu.sync_copy(gx_hbm.at[pl.ds(rbase, RCH)], gxf)
        pltpu.sync_copy(gy_hbm.at[pl.ds(rbase, RCH)], gyf)

        zero = jnp.zeros((L,), jnp.int32)

        def zfill(k, carry):
            zbuf[pl.ds(k * L, L)] = zero
            return carry

        lax.fori_loop(0, RCH // L, zfill, 0)
        for t in range(STRIPE // RCH):
            pltpu.sync_copy(zbuf, table.at[pl.ds(sid * STRIPE + t * RCH, RCH)])

        one = jnp.ones((L,), jnp.int32)

        def kfill(k, carry):
            sl = pl.ds(k * L, L)
            keys[sl] = gxf[sl] * M + gyf[sl]
            vbuf[sl] = (BASE + rbase + k * L) + iota
            act[sl] = one
            return carry

        lax.fori_loop(0, RCH // L, kfill, 0)
        plsc.subcore_barrier()

        pltpu.sync_copy(vbuf, table.at[keys], add=True)
        plsc.subcore_barrier()

        for _ in range(ROUNDS):
            pltpu.sync_copy(table.at[keys], gbuf)

            def rbody(k, carry):
                sl = pl.ds(k * L, L)
                v = gbuf[sl]
                a = act[sl]
                cnt = lax.shift_right_logical(v, 26)
                ssum = v & MASK
                i_vec = (rbase + k * L) + iota
                elim = (a != 0) & (cnt * i_vec < ssum)
                vbuf[sl] = jnp.where(elim, -BASE - i_vec, 0)
                act[sl] = jnp.where(elim, 0, a)
                return carry

            lax.fori_loop(0, RCH // L, rbody, 0)
            plsc.subcore_barrier()
            pltpu.sync_copy(vbuf, table.at[keys], add=True)
            plsc.subcore_barrier()

        wid = sid * NC + cid
        pltpu.sync_copy(table.at[keys.at[pl.ds(cid * GCH, GCH)]], widx)

        def wbody(k, carry):
            sl = pl.ds(k * L, L)
            widx[sl] = widx[sl] & MASK
            return carry

        lax.fori_loop(0, GCH // L, wbody, 0)

        obase = wid * GCH
        sems = (sem0, sem1)
        desc = pltpu.async_copy(upd_hbm.at[widx.at[pl.ds(0, 128)]],
                                rowbuf.at[0], sems[0])
        for w in range(GROWS):
            nxt = None
            if w + 1 < GROWS:
                nxt = pltpu.async_copy(
                    upd_hbm.at[widx.at[pl.ds((w + 1) * 128, 128)]],
                    rowbuf.at[(w + 1) % 2], sems[(w + 1) % 2])
            desc.wait()
            pltpu.sync_copy(rowbuf.at[w % 2],
                            out_hbm.at[pl.ds(obase + w * 128, 128)])
            desc = nxt

    return pl.kernel(
        body,
        out_type=jax.ShapeDtypeStruct((B, H), jnp.float32),
        mesh=mesh,
        compiler_params=pltpu.CompilerParams(use_tc_tiling_on_sc=False),
        scratch_types=[
            pltpu.VMEM_SHARED((NM,), jnp.int32),
            pltpu.VMEM((RCH,), jnp.int32),
            pltpu.VMEM((RCH,), jnp.int32),
            pltpu.VMEM((RCH,), jnp.int32),
            pltpu.VMEM((RCH,), jnp.int32),
            pltpu.VMEM((RCH,), jnp.int32),
            pltpu.VMEM((RCH,), jnp.int32),
            pltpu.VMEM((GCH,), jnp.int32),
            pltpu.VMEM((RCH,), jnp.int32),
            pltpu.VMEM((2, 128, H), jnp.float32),
            pltpu.SemaphoreType.DMA,
            pltpu.SemaphoreType.DMA,
        ],
    )


def kernel(mem, grid_x, grid_y, updates):
    N, M, H = mem.shape
    B = grid_x.shape[0]
    del mem
    return _build(N, M, H, B)(grid_x, grid_y, updates)
